# baseline (speedup 1.0000x reference)
import functools

import jax
import jax.numpy as jnp
from jax import lax
from jax.experimental import pallas as pl
from jax.experimental.pallas import tpu as pltpu
from jax.experimental.pallas import tpu_sc as plsc

N_REL = 64
ED = 64
RD = 64
B = 4096
L2_LAMBDA = 1e-05

G = 8
GE = G * ED

NW = 32
NG = 3 * B
GPW = NG // NW
GC = 128

BB = 2048
NB = B // BB

DQ = 3200
NSTEP = 8
PACKED = NSTEP * DQ
HED = ED // 2


def _depad_body(x_ref, o_ref):
    xt = jnp.transpose(x_ref[...], (1, 0))
    xtb = xt.astype(jnp.bfloat16)
    for q in range(4):
        seg = xtb[q * DQ:(q + 1) * DQ, :]
        lo = lax.bitcast_convert_type(seg[:, 0:HED], jnp.uint16)
        hi = lax.bitcast_convert_type(seg[:, HED:ED], jnp.uint16)
        w = lo.astype(jnp.uint32) | (hi.astype(jnp.uint32) << 16)
        o_ref[:, HED * q:HED * (q + 1)] = lax.bitcast_convert_type(
            w, jnp.float32)


def _depad(entity_t):
    return pl.pallas_call(
        _depad_body,
        grid=(NSTEP,),
        in_specs=[pl.BlockSpec((ED, 4 * DQ), lambda i: (0, i))],
        out_specs=pl.BlockSpec((DQ, 4 * HED), lambda i: (i, 0)),
        out_shape=jax.ShapeDtypeStruct((PACKED, 4 * HED), jnp.float32),
    )(entity_t)


def _sc_gather(ent_pairs, idx_packed):
    mesh = plsc.VectorSubcoreMesh(core_axis_name="c", subcore_axis_name="s")

    @functools.partial(
        pl.kernel,
        out_type=jax.ShapeDtypeStruct((NG, 2 * ED), jnp.float32),
        mesh=mesh,
        scratch_types=[
            pltpu.VMEM((GPW,), jnp.int32),
            pltpu.VMEM((GC, 2 * ED), jnp.float32),
            pltpu.SemaphoreType.DMA,
        ],
        compiler_params=pltpu.CompilerParams(use_tc_tiling_on_sc=True),
    )
    def gather_k(ent_hbm, idx_hbm, out_hbm, idx_v, rows_v, sem):
        wid = lax.axis_index("s") * 2 + lax.axis_index("c")
        base = wid * GPW
        pltpu.sync_copy(idx_hbm.at[pl.ds(base, GPW)], idx_v)
        for c in range(GPW // GC):
            pltpu.async_copy(ent_hbm.at[idx_v.at[pl.ds(c * GC, GC)]],
                             rows_v, sem).wait()
            pltpu.sync_copy(rows_v, out_hbm.at[pl.ds(base + c * GC, GC)])

    return gather_k(ent_pairs, idx_packed)


def _tc_body(h_ref, p_ref, n_ref, hp_ref, pp_ref, np_ref, rel_ref, r_ref,
             wg_ref, out_ref, acc_ref):
    @pl.when(pl.program_id(0) == 0)
    def _init():
        acc_ref[0] = 0.0
        acc_ref[1] = 0.0

    wg = wg_ref[...]
    rcol = jnp.transpose(r_ref[0], (1, 0))
    r1 = rcol // G
    r0 = rcol % G
    lane_g = lax.broadcasted_iota(jnp.int32, (1, GE), 1) // ED
    mask1 = lane_g == r1
    mask0 = lane_g == r0
    zero16 = jnp.zeros((), jnp.bfloat16)
    f0 = (lax.broadcasted_iota(jnp.int32, (GE, RD), 0) % ED
          == lax.broadcasted_iota(jnp.int32, (GE, RD), 1)).astype(jnp.float32)

    def proj(pair_ref, par_ref):
        pair = pair_ref[...]
        q = jnp.transpose(par_ref[0], (1, 0))
        half = jnp.where(q >= 2, pair[:, 2 * HED:4 * HED], pair[:, 0:2 * HED])
        quart = jnp.where(q % 2 == 1, half[:, HED:2 * HED], half[:, 0:HED])
        u = lax.bitcast_convert_type(quart, jnp.uint32)
        lo = lax.bitcast_convert_type(
            (u & 0xFFFF).astype(jnp.uint16), jnp.bfloat16)
        hi = lax.bitcast_convert_type(
            (u >> 16).astype(jnp.uint16), jnp.bfloat16)
        x = jnp.concatenate([lo, hi], axis=1)
        xt = jnp.tile(x, (1, G))
        x1 = jnp.where(mask1, xt, zero16)
        y = lax.dot_general(x1, wg, (((1,), (0,)), ((), ())),
                            preferred_element_type=jnp.float32)
        ys = jnp.where(mask0, y, 0.0)
        return lax.dot_general(ys, f0, (((1,), (0,)), ((), ())),
                               preferred_element_type=jnp.float32)

    rh = proj(h_ref, hp_ref)
    rp = proj(p_ref, pp_ref)
    rn = proj(n_ref, np_ref)

    lane_r = lax.broadcasted_iota(jnp.int32, (1, N_REL), 1)
    oh = (lane_r == rcol).astype(jnp.float32)
    re = lax.dot_general(oh, rel_ref[...], (((1,), (0,)), ((), ())),
                         preferred_element_type=jnp.float32)

    anchor = rh + re
    pos_s = jnp.sum(jnp.square(anchor - rp), axis=1, keepdims=True)
    neg_s = jnp.sum(jnp.square(anchor - rn), axis=1, keepdims=True)
    d = neg_s - pos_s
    trip = jnp.maximum(-d, 0.0) + jnp.log(1.0 + jnp.exp(-jnp.abs(d)))
    l2 = 0.5 * (jnp.sum(jnp.square(rh)) + jnp.sum(jnp.square(re))
                + jnp.sum(jnp.square(rp)) + jnp.sum(jnp.square(rn)))

    acc_ref[0] += jnp.sum(trip)
    acc_ref[1] += l2

    @pl.when(pl.program_id(0) == NB - 1)
    def _fin():
        loss = acc_ref[0] / B + L2_LAMBDA * (acc_ref[1] / B)
        out_ref[...] = jnp.full((1, 1), loss, dtype=jnp.float32)


def _tc_loss(ent_pairs_rows, parity, relation_embed, r_rows, wg_16):
    ent_spec = lambda a: pl.BlockSpec((BB, 2 * ED), lambda i, a=a: (i + a * NB, 0))
    par_spec = lambda a: pl.BlockSpec((1, 1, BB), lambda i, a=a: (a * NB + i, 0, 0))
    return pl.pallas_call(
        _tc_body,
        grid=(NB,),
        in_specs=[
            ent_spec(0), ent_spec(1), ent_spec(2),
            par_spec(0), par_spec(1), par_spec(2),
            pl.BlockSpec((N_REL, RD), lambda i: (0, 0)),
            pl.BlockSpec((1, 1, BB), lambda i: (i, 0, 0)),
            pl.BlockSpec((GE, GE), lambda i: (0, 0)),
        ],
        out_specs=pl.BlockSpec((1, 1), lambda i: (0, 0)),
        out_shape=jax.ShapeDtypeStruct((1, 1), jnp.float32),
        scratch_shapes=[pltpu.SMEM((2,), jnp.float32)],
        compiler_params=pltpu.CompilerParams(
            dimension_semantics=("arbitrary",)),
    )(ent_pairs_rows, ent_pairs_rows, ent_pairs_rows,
      parity, parity, parity, relation_embed, r_rows, wg_16)


def _pack_index(idx):
    step = idx // (4 * DQ)
    rin = idx % (4 * DQ)
    q = rin // DQ
    packed = step * DQ + rin % DQ
    return packed, q


def kernel(h, r, pos_t, neg_t, entity_embed, relation_embed, trans_M):
    h = h.astype(jnp.int32)
    r = r.astype(jnp.int32)
    pos_t = pos_t.astype(jnp.int32)
    neg_t = neg_t.astype(jnp.int32)
    idx_all = jnp.concatenate([h, pos_t, neg_t])
    ent_pairs = _depad(entity_embed.T)
    idx_packed, par = _pack_index(idx_all)
    rows = _sc_gather(ent_pairs, idx_packed)
    parity = par.reshape(3 * NB, 1, BB)
    wg_16 = (trans_M.reshape(G, G, ED, RD).transpose(0, 2, 1, 3)
             .reshape(GE, GE).astype(jnp.bfloat16))
    out = _tc_loss(rows, parity, relation_embed,
                   r.reshape(NB, 1, BB), wg_16)
    return out[0, 0]


# speedup vs baseline: 1.0725x; 1.0725x over previous
import functools

import jax
import jax.numpy as jnp
from jax import lax
from jax.experimental import pallas as pl
from jax.experimental.pallas import tpu as pltpu
from jax.experimental.pallas import tpu_sc as plsc

N_REL = 64
ED = 64
RD = 64
B = 4096
L2_LAMBDA = 1e-05

G = 8
GE = G * ED

NW = 32
NG = 3 * B
GPW = NG // NW
GC = 128

BB = 2048
NB = B // BB

PACKED = 51200
DP = 6400


def _depad_body(x_ref, o_ref):
    x = jnp.transpose(x_ref[...], (1, 0))
    o_ref[...] = jnp.concatenate([x[:DP, :], x[DP:, :]], axis=1)


def _depad(entity_t):
    return pl.pallas_call(
        _depad_body,
        grid=(PACKED // DP,),
        in_specs=[pl.BlockSpec((ED, 2 * DP), lambda i: (0, i))],
        out_specs=pl.BlockSpec((DP, 2 * ED), lambda i: (i, 0)),
        out_shape=jax.ShapeDtypeStruct((PACKED, 2 * ED), jnp.float32),
    )(entity_t)


def _sc_gather(ent_pairs, idx_packed):
    mesh = plsc.VectorSubcoreMesh(core_axis_name="c", subcore_axis_name="s")

    @functools.partial(
        pl.kernel,
        out_type=jax.ShapeDtypeStruct((NG, 2 * ED), jnp.float32),
        mesh=mesh,
        scratch_types=[
            pltpu.VMEM((GPW,), jnp.int32),
            pltpu.VMEM((GC, 2 * ED), jnp.float32),
            pltpu.SemaphoreType.DMA,
        ],
        compiler_params=pltpu.CompilerParams(use_tc_tiling_on_sc=True),
    )
    def gather_k(ent_hbm, idx_hbm, out_hbm, idx_v, rows_v, sem):
        wid = lax.axis_index("s") * 2 + lax.axis_index("c")
        base = wid * GPW
        pltpu.sync_copy(idx_hbm.at[pl.ds(base, GPW)], idx_v)
        for c in range(GPW // GC):
            pltpu.async_copy(ent_hbm.at[idx_v.at[pl.ds(c * GC, GC)]],
                             rows_v, sem).wait()
            pltpu.sync_copy(rows_v, out_hbm.at[pl.ds(base + c * GC, GC)])

    return gather_k(ent_pairs, idx_packed)


def _tc_body(h_ref, p_ref, n_ref, hp_ref, pp_ref, np_ref, rel_ref, r_ref,
             wg_ref, out_ref, acc_ref):
    @pl.when(pl.program_id(0) == 0)
    def _init():
        acc_ref[0] = 0.0
        acc_ref[1] = 0.0

    wg = wg_ref[...]
    rcol = jnp.transpose(r_ref[0], (1, 0))
    r1 = rcol // G
    r0 = rcol % G
    lane_g = lax.broadcasted_iota(jnp.int32, (1, GE), 1) // ED
    mask1 = lane_g == r1
    mask0 = lane_g == r0
    zero16 = jnp.zeros((), jnp.bfloat16)
    f0 = (lax.broadcasted_iota(jnp.int32, (GE, RD), 0) % ED
          == lax.broadcasted_iota(jnp.int32, (GE, RD), 1)).astype(jnp.float32)

    def proj(pair_ref, par_ref):
        pair = pair_ref[...]
        par = jnp.transpose(par_ref[0], (1, 0))
        x = jnp.where(par == 1, pair[:, ED:2 * ED], pair[:, 0:ED])
        xt = jnp.tile(x.astype(jnp.bfloat16), (1, G))
        x1 = jnp.where(mask1, xt, zero16)
        y = lax.dot_general(x1, wg, (((1,), (0,)), ((), ())),
                            preferred_element_type=jnp.float32)
        ys = jnp.where(mask0, y, 0.0)
        return lax.dot_general(ys, f0, (((1,), (0,)), ((), ())),
                               preferred_element_type=jnp.float32)

    rh = proj(h_ref, hp_ref)
    rp = proj(p_ref, pp_ref)
    rn = proj(n_ref, np_ref)

    lane_r = lax.broadcasted_iota(jnp.int32, (1, N_REL), 1)
    oh = (lane_r == rcol).astype(jnp.float32)
    re = lax.dot_general(oh, rel_ref[...], (((1,), (0,)), ((), ())),
                         preferred_element_type=jnp.float32)

    anchor = rh + re
    pos_s = jnp.sum(jnp.square(anchor - rp), axis=1, keepdims=True)
    neg_s = jnp.sum(jnp.square(anchor - rn), axis=1, keepdims=True)
    d = neg_s - pos_s
    trip = jnp.maximum(-d, 0.0) + jnp.log(1.0 + jnp.exp(-jnp.abs(d)))
    l2 = 0.5 * (jnp.sum(jnp.square(rh)) + jnp.sum(jnp.square(re))
                + jnp.sum(jnp.square(rp)) + jnp.sum(jnp.square(rn)))

    acc_ref[0] += jnp.sum(trip)
    acc_ref[1] += l2

    @pl.when(pl.program_id(0) == NB - 1)
    def _fin():
        loss = acc_ref[0] / B + L2_LAMBDA * (acc_ref[1] / B)
        out_ref[...] = jnp.full((1, 1), loss, dtype=jnp.float32)


def _tc_loss(ent_pairs_rows, parity, relation_embed, r_rows, wg_16):
    ent_spec = lambda a: pl.BlockSpec((BB, 2 * ED), lambda i, a=a: (i + a * NB, 0))
    par_spec = lambda a: pl.BlockSpec((1, 1, BB), lambda i, a=a: (a * NB + i, 0, 0))
    return pl.pallas_call(
        _tc_body,
        grid=(NB,),
        in_specs=[
            ent_spec(0), ent_spec(1), ent_spec(2),
            par_spec(0), par_spec(1), par_spec(2),
            pl.BlockSpec((N_REL, RD), lambda i: (0, 0)),
            pl.BlockSpec((1, 1, BB), lambda i: (i, 0, 0)),
            pl.BlockSpec((GE, GE), lambda i: (0, 0)),
        ],
        out_specs=pl.BlockSpec((1, 1), lambda i: (0, 0)),
        out_shape=jax.ShapeDtypeStruct((1, 1), jnp.float32),
        scratch_shapes=[pltpu.SMEM((2,), jnp.float32)],
        compiler_params=pltpu.CompilerParams(
            dimension_semantics=("arbitrary",)),
    )(ent_pairs_rows, ent_pairs_rows, ent_pairs_rows,
      parity, parity, parity, relation_embed, r_rows, wg_16)


def _pack_index(idx):
    step = idx // (2 * DP)
    rin = idx % (2 * DP)
    par = (rin >= DP).astype(jnp.int32)
    packed = step * DP + rin - par * DP
    return packed, par


def kernel(h, r, pos_t, neg_t, entity_embed, relation_embed, trans_M):
    h = h.astype(jnp.int32)
    r = r.astype(jnp.int32)
    pos_t = pos_t.astype(jnp.int32)
    neg_t = neg_t.astype(jnp.int32)
    idx_all = jnp.concatenate([h, pos_t, neg_t])
    ent_pairs = _depad(entity_embed.T)
    idx_packed, par = _pack_index(idx_all)
    rows = _sc_gather(ent_pairs, idx_packed)
    parity = par.reshape(3 * NB, 1, BB)
    wg_16 = (trans_M.reshape(G, G, ED, RD).transpose(0, 2, 1, 3)
             .reshape(GE, GE).astype(jnp.bfloat16))
    out = _tc_loss(rows, parity, relation_embed,
                   r.reshape(NB, 1, BB), wg_16)
    return out[0, 0]
